# baseline (speedup 1.0000x reference)
import jax
import jax.numpy as jnp
from jax import lax
from jax.experimental import pallas as pl
from jax.experimental.pallas import tpu as pltpu
from jax.experimental.pallas import tpu_sc as plsc

BATCH = 16384
DIM = 64
NC = 2
NS = 16
NW = NC * NS
B_PER_W = BATCH // NW
GROUPS = B_PER_W // 16


def _distmult_body(hidx_hbm, ridx_hbm, oidx_hbm, ent_hbm, rel_hbm, out_hbm,
                   hidx_v, ridx_v, oidx_v, h_rows, r_rows, o_rows, out_v,
                   sem_h, sem_r, sem_o):
    wid = lax.axis_index("s") * NC + lax.axis_index("c")
    base = wid * B_PER_W

    pltpu.sync_copy(hidx_hbm.at[pl.ds(base, B_PER_W)], hidx_v)
    pltpu.sync_copy(ridx_hbm.at[pl.ds(base, B_PER_W)], ridx_v)
    pltpu.sync_copy(oidx_hbm.at[pl.ds(base, B_PER_W)], oidx_v)

    cp_h = pltpu.async_copy(ent_hbm.at[hidx_v], h_rows, sem_h)
    cp_r = pltpu.async_copy(rel_hbm.at[ridx_v], r_rows, sem_r)
    cp_o = pltpu.async_copy(ent_hbm.at[oidx_v], o_rows, sem_o)
    cp_h.wait()
    cp_r.wait()
    cp_o.wait()

    iota16 = lax.iota(jnp.int32, 16)

    def lanesum(v):
        for s in (8, 4, 2, 1):
            v = v + jnp.take_along_axis(v, iota16 ^ s, axis=0,
                                        mode="promise_in_bounds")
        return v

    def group(g, carry):
        acc = jnp.zeros((16,), jnp.float32)
        for i in range(16):
            row = g * 16 + i
            p = jnp.zeros((16,), jnp.float32)
            for k in range(DIM // 16):
                sl = pl.ds(k * 16, 16)
                p = p + h_rows[row, sl] * r_rows[row, sl] * o_rows[row, sl]
            acc = jnp.where(iota16 == i, lanesum(p), acc)
        out_v[pl.ds(g * 16, 16)] = acc
        return carry

    lax.fori_loop(0, GROUPS, group, 0)
    pltpu.sync_copy(out_v, out_hbm.at[pl.ds(base, B_PER_W)])


def kernel(triplets, ent_emb, rel_emb):
    hidx = triplets[:, 0].astype(jnp.int32)
    ridx = triplets[:, 1].astype(jnp.int32)
    oidx = triplets[:, 2].astype(jnp.int32)

    mesh = plsc.VectorSubcoreMesh(core_axis_name="c", subcore_axis_name="s")
    score = pl.kernel(
        _distmult_body,
        out_type=jax.ShapeDtypeStruct((BATCH,), jnp.float32),
        mesh=mesh,
        scratch_types=[
            pltpu.VMEM((B_PER_W,), jnp.int32),
            pltpu.VMEM((B_PER_W,), jnp.int32),
            pltpu.VMEM((B_PER_W,), jnp.int32),
            pltpu.VMEM((B_PER_W, DIM), jnp.float32),
            pltpu.VMEM((B_PER_W, DIM), jnp.float32),
            pltpu.VMEM((B_PER_W, DIM), jnp.float32),
            pltpu.VMEM((B_PER_W,), jnp.float32),
            pltpu.SemaphoreType.DMA,
            pltpu.SemaphoreType.DMA,
            pltpu.SemaphoreType.DMA,
        ],
        compiler_params=pltpu.CompilerParams(use_tc_tiling_on_sc=False),
    )(hidx, ridx, oidx, ent_emb, rel_emb)
    return score


# speedup vs baseline: 1.6053x; 1.6053x over previous
import jax
import jax.numpy as jnp
from jax import lax
from jax.experimental import pallas as pl
from jax.experimental.pallas import tpu as pltpu
from jax.experimental.pallas import tpu_sc as plsc

BATCH = 16384
DIM = 64
NC = 2
NS = 16
NW = NC * NS
B_PER_W = BATCH // NW
CH = 128
NCHUNK = B_PER_W // CH
CH_WORDS = CH * DIM

_TREE_PERM = (0, 8, 4, 12, 2, 10, 6, 14, 1, 9, 5, 13, 3, 11, 7, 15)


def _distmult_body(hidx_hbm, ridx_hbm, oidx_hbm, ent_hbm, rel_hbm, out_hbm,
                   hidx_v, ridx_v, oidx_v,
                   h0, r0, o0, h1, r1, o1, out_v, sem0, sem1):
    wid = lax.axis_index("s") * NC + lax.axis_index("c")
    base = wid * B_PER_W

    pltpu.sync_copy(hidx_hbm.at[pl.ds(base, B_PER_W)], hidx_v)
    pltpu.sync_copy(ridx_hbm.at[pl.ds(base, B_PER_W)], ridx_v)
    pltpu.sync_copy(oidx_hbm.at[pl.ds(base, B_PER_W)], oidx_v)

    sems = (sem0, sem1)
    slots = ((h0, r0, o0), (h1, r1, o1))
    iota16 = lax.iota(jnp.int32, 16)
    perm = (((iota16 & 1) << 3) | ((iota16 & 2) << 1)
            | ((iota16 & 4) >> 1) | ((iota16 & 8) >> 3))

    def fire(c, slot):
        sem = sems[slot]

        def body(jj, carry):
            hvec = hidx_v[pl.ds(c * CH + jj * 16, 16)]
            rvec = ridx_v[pl.ds(c * CH + jj * 16, 16)]
            ovec = oidx_v[pl.ds(c * CH + jj * 16, 16)]
            hb, rb, ob = slots[slot]
            for i in range(16):
                j = jj * 16 + i
                pltpu.async_copy(ent_hbm.at[hvec[i]], hb.at[j], sem)
                pltpu.async_copy(rel_hbm.at[rvec[i]], rb.at[j], sem)
                pltpu.async_copy(ent_hbm.at[ovec[i]], ob.at[j], sem)
            return carry

        lax.fori_loop(0, CH // 16, body, 0)

    def drain(slot):
        for t in range(3):
            pltpu.make_async_copy(
                ent_hbm.at[pl.ds(0, CH), :], slots[slot][t], sems[slot]
            ).wait()

    def ptake(v, idx):
        return jnp.take_along_axis(v, idx, axis=0, mode="promise_in_bounds")

    def compute(c, slot):
        hb, rb, ob = slots[slot]

        def merge(a, b, s):
            a1 = a + ptake(a, iota16 ^ s)
            b1 = b + ptake(b, iota16 ^ s)
            return jnp.where((iota16 & s) == 0, a1, ptake(b1, iota16 ^ s))

        def group(g, carry):
            stack = []
            for i in range(16):
                row = g * 16 + i
                p = jnp.zeros((16,), jnp.float32)
                for k in range(DIM // 16):
                    sl = pl.ds(k * 16, 16)
                    p = p + hb[row, sl] * rb[row, sl] * ob[row, sl]
                node = (0, p)
                while stack and stack[-1][0] == node[0]:
                    lvl, a = stack.pop()
                    node = (lvl + 1, merge(a, node[1], 16 >> (lvl + 1)))
                stack.append(node)
            out_v[pl.ds(c * CH + g * 16, 16)] = ptake(stack[0][1], perm)
            return carry

        lax.fori_loop(0, CH // 16, group, 0)

    fire(0, 0)
    for c in range(NCHUNK):
        slot = c % 2
        if c + 1 < NCHUNK:
            fire(c + 1, 1 - slot)
        drain(slot)
        compute(c, slot)

    pltpu.sync_copy(out_v, out_hbm.at[pl.ds(base, B_PER_W)])


def kernel(triplets, ent_emb, rel_emb):
    hidx = triplets[:, 0].astype(jnp.int32)
    ridx = triplets[:, 1].astype(jnp.int32)
    oidx = triplets[:, 2].astype(jnp.int32)

    mesh = plsc.VectorSubcoreMesh(core_axis_name="c", subcore_axis_name="s")
    score = pl.kernel(
        _distmult_body,
        out_type=jax.ShapeDtypeStruct((BATCH,), jnp.float32),
        mesh=mesh,
        scratch_types=[
            pltpu.VMEM((B_PER_W,), jnp.int32),
            pltpu.VMEM((B_PER_W,), jnp.int32),
            pltpu.VMEM((B_PER_W,), jnp.int32),
            pltpu.VMEM((CH, DIM), jnp.float32),
            pltpu.VMEM((CH, DIM), jnp.float32),
            pltpu.VMEM((CH, DIM), jnp.float32),
            pltpu.VMEM((CH, DIM), jnp.float32),
            pltpu.VMEM((CH, DIM), jnp.float32),
            pltpu.VMEM((CH, DIM), jnp.float32),
            pltpu.VMEM((B_PER_W,), jnp.float32),
            pltpu.SemaphoreType.DMA,
            pltpu.SemaphoreType.DMA,
        ],
    )(hidx, ridx, oidx, ent_emb, rel_emb)
    return score
